# baseline (speedup 1.0000x reference)
import jax
import jax.numpy as jnp
from jax.experimental import pallas as pl

_NL = 128
_J = 12
_NV = 4
_NC = 8
_NM = 16


def _kern(x_ref, cl_ref, ct_ref, ml_ref, mt_ref, sig_ref, out_ref):
    s = jnp.maximum(sig_ref[0, 0], 1e-10)
    h = -0.5 / (s * s)
    cl = cl_ref[...]
    ct = ct_ref[...]
    ml = ml_ref[...]
    mt = mt_ref[...]

    ws = []
    denom = None
    for j in range(_J):
        a = cl[j:j + 1, :] - ml
        b = ct[j:j + 1, :] - mt
        wj = jnp.exp(a * a * h) * jnp.exp(b * b * h)
        ws.append(wj)
        denom = wj if denom is None else denom + wj

    acc = None
    for j in range(_J):
        wn = ws[j] / denom
        xj = x_ref[j * 32:(j + 1) * 32, :].reshape(_NV, _NC, -1)
        t = wn[None, :, None, :] * xj[:, None, :, :]
        acc = t if acc is None else acc + t
    out_ref[...] = acc.reshape(_NV * _NM * _NC, -1)


def kernel(x, coords_lon, coords_lat, mus_lon, mus_lat, sigma):
    b, n, seq_ref, seq_in, nh_in = coords_lon.shape
    nv, nc = x.shape[-2], x.shape[-1]
    n_lon, n_lat = mus_lon.shape[0], mus_lat.shape[0]
    j = seq_in * nh_in
    n_mu = n_lon * n_lat

    xt = x.reshape(n, j * nv * nc).T
    clt = coords_lon.reshape(n, j).T
    ctt = coords_lat.reshape(n, j).T
    ml16 = jnp.repeat(mus_lon, n_lat).reshape(n_mu, 1)
    mt16 = jnp.tile(mus_lat, n_lon).reshape(n_mu, 1)
    sig = jnp.asarray(sigma, jnp.float32).reshape(1, 1)

    grid = (pl.cdiv(n, _NL),)
    out = pl.pallas_call(
        _kern,
        grid=grid,
        in_specs=[
            pl.BlockSpec((j * nv * nc, _NL), lambda i: (0, i)),
            pl.BlockSpec((j, _NL), lambda i: (0, i)),
            pl.BlockSpec((j, _NL), lambda i: (0, i)),
            pl.BlockSpec((n_mu, 1), lambda i: (0, 0)),
            pl.BlockSpec((n_mu, 1), lambda i: (0, 0)),
            pl.BlockSpec((1, 1), lambda i: (0, 0)),
        ],
        out_specs=pl.BlockSpec((nv * n_mu * nc, _NL), lambda i: (0, i)),
        out_shape=jax.ShapeDtypeStruct((nv * n_mu * nc, n), jnp.float32),
    )(xt, clt, ctt, ml16, mt16, sig)
    return out.reshape(nv, n_lon, n_lat, nc, n).transpose(4, 0, 1, 2, 3)[None]


# speedup vs baseline: 1.1835x; 1.1835x over previous
import jax
import jax.numpy as jnp
from jax.experimental import pallas as pl

_NL = 1024
_J = 12
_NV = 4
_NC = 8
_NM = 16


def _kern(x_ref, cl_ref, ct_ref, ml_ref, mt_ref, sig_ref, out_ref):
    s = jnp.maximum(sig_ref[0, 0], 1e-10)
    h = -0.5 / (s * s)
    cl = cl_ref[...]
    ct = ct_ref[...]
    ml = ml_ref[...]
    mt = mt_ref[...]

    ws = []
    denom = None
    for j in range(_J):
        a = cl[j:j + 1, :] - ml
        b = ct[j:j + 1, :] - mt
        wj = jnp.exp(a * a * h) * jnp.exp(b * b * h)
        ws.append(wj)
        denom = wj if denom is None else denom + wj

    acc = None
    for j in range(_J):
        wn = ws[j] / denom
        xj = x_ref[j * 32:(j + 1) * 32, :].reshape(_NV, _NC, -1)
        t = wn[None, :, None, :] * xj[:, None, :, :]
        acc = t if acc is None else acc + t
    out_ref[...] = acc.reshape(_NV * _NM * _NC, -1)


def kernel(x, coords_lon, coords_lat, mus_lon, mus_lat, sigma):
    b, n, seq_ref, seq_in, nh_in = coords_lon.shape
    nv, nc = x.shape[-2], x.shape[-1]
    n_lon, n_lat = mus_lon.shape[0], mus_lat.shape[0]
    j = seq_in * nh_in
    n_mu = n_lon * n_lat

    xt = x.reshape(n, j * nv * nc).T
    clt = coords_lon.reshape(n, j).T
    ctt = coords_lat.reshape(n, j).T
    ml16 = jnp.repeat(mus_lon, n_lat).reshape(n_mu, 1)
    mt16 = jnp.tile(mus_lat, n_lon).reshape(n_mu, 1)
    sig = jnp.asarray(sigma, jnp.float32).reshape(1, 1)

    grid = (pl.cdiv(n, _NL),)
    out = pl.pallas_call(
        _kern,
        grid=grid,
        in_specs=[
            pl.BlockSpec((j * nv * nc, _NL), lambda i: (0, i)),
            pl.BlockSpec((j, _NL), lambda i: (0, i)),
            pl.BlockSpec((j, _NL), lambda i: (0, i)),
            pl.BlockSpec((n_mu, 1), lambda i: (0, 0)),
            pl.BlockSpec((n_mu, 1), lambda i: (0, 0)),
            pl.BlockSpec((1, 1), lambda i: (0, 0)),
        ],
        out_specs=pl.BlockSpec((nv * n_mu * nc, _NL), lambda i: (0, i)),
        out_shape=jax.ShapeDtypeStruct((nv * n_mu * nc, n), jnp.float32),
    )(xt, clt, ctt, ml16, mt16, sig)
    return out.reshape(nv, n_lon, n_lat, nc, n).transpose(4, 0, 1, 2, 3)[None]


# speedup vs baseline: 1.2565x; 1.0617x over previous
import jax
import jax.numpy as jnp
from jax.experimental import pallas as pl
from jax.experimental.pallas import tpu as pltpu

_NL = 512
_J = 12
_NV = 4
_NC = 8
_NM = 16


def _kern(x_ref, cl_ref, ct_ref, ml_ref, mt_ref, sig_ref, out_ref):
    s = jnp.maximum(sig_ref[0, 0], 1e-10)
    h = -0.5 / (s * s)
    cl = cl_ref[...]
    ct = ct_ref[...]
    ml = ml_ref[...]
    mt = mt_ref[...]

    ws = []
    denom = None
    for j in range(_J):
        a = cl[j:j + 1, :] - ml
        b = ct[j:j + 1, :] - mt
        wj = jnp.exp(a * a * h) * jnp.exp(b * b * h)
        ws.append(wj)
        denom = wj if denom is None else denom + wj

    acc = None
    for j in range(_J):
        wn = ws[j] / denom
        xj = x_ref[j * 32:(j + 1) * 32, :].reshape(_NV, _NC, -1)
        t = wn[None, :, None, :] * xj[:, None, :, :]
        acc = t if acc is None else acc + t
    out_ref[...] = acc.reshape(_NV * _NM * _NC, -1)


def kernel(x, coords_lon, coords_lat, mus_lon, mus_lat, sigma):
    b, n, seq_ref, seq_in, nh_in = coords_lon.shape
    nv, nc = x.shape[-2], x.shape[-1]
    n_lon, n_lat = mus_lon.shape[0], mus_lat.shape[0]
    j = seq_in * nh_in
    n_mu = n_lon * n_lat

    xt = x.reshape(n, j * nv * nc).T
    clt = coords_lon.reshape(n, j).T
    ctt = coords_lat.reshape(n, j).T
    ml16 = jnp.repeat(mus_lon, n_lat).reshape(n_mu, 1)
    mt16 = jnp.tile(mus_lat, n_lon).reshape(n_mu, 1)
    sig = jnp.asarray(sigma, jnp.float32).reshape(1, 1)

    grid = (pl.cdiv(n, _NL),)
    out = pl.pallas_call(
        _kern,
        grid=grid,
        in_specs=[
            pl.BlockSpec((j * nv * nc, _NL), lambda i: (0, i)),
            pl.BlockSpec((j, _NL), lambda i: (0, i)),
            pl.BlockSpec((j, _NL), lambda i: (0, i)),
            pl.BlockSpec((n_mu, 1), lambda i: (0, 0)),
            pl.BlockSpec((n_mu, 1), lambda i: (0, 0)),
            pl.BlockSpec((1, 1), lambda i: (0, 0)),
        ],
        out_specs=pl.BlockSpec((nv * n_mu * nc, _NL), lambda i: (0, i)),
        out_shape=jax.ShapeDtypeStruct((nv * n_mu * nc, n), jnp.float32),
        compiler_params=pltpu.CompilerParams(
            dimension_semantics=("parallel",)),
    )(xt, clt, ctt, ml16, mt16, sig)
    return out.reshape(nv, n_lon, n_lat, nc, n).transpose(4, 0, 1, 2, 3)[None]


# speedup vs baseline: 1.3757x; 1.0948x over previous
import jax
import jax.numpy as jnp
from jax.experimental import pallas as pl
from jax.experimental.pallas import tpu as pltpu

_NL = 512
_J = 12
_NV = 4
_NC = 8
_NM = 16


def _kern(x_ref, cl_ref, ct_ref, ml_ref, mt_ref, sig_ref, out_ref):
    s = jnp.maximum(sig_ref[0, 0], 1e-10)
    h = -0.5 / (s * s)
    cl = cl_ref[...]
    ct = ct_ref[...]
    ml = ml_ref[...]
    mt = mt_ref[...]

    ws = []
    denom = None
    for j in range(_J):
        a = cl[j:j + 1, :] - ml
        b = ct[j:j + 1, :] - mt
        wj = jnp.exp(a * a * h) * jnp.exp(b * b * h)
        ws.append(wj)
        denom = wj if denom is None else denom + wj

    wn = [w / denom for w in ws]
    for v in range(_NV):
        acc = None
        for j in range(_J):
            xj = x_ref[j * 32 + v * _NC:j * 32 + (v + 1) * _NC, :]
            t = wn[j][:, None, :] * xj[None, :, :]
            acc = t if acc is None else acc + t
        out_ref[v * _NM * _NC:(v + 1) * _NM * _NC, :] = acc.reshape(
            _NM * _NC, -1)


def kernel(x, coords_lon, coords_lat, mus_lon, mus_lat, sigma):
    b, n, seq_ref, seq_in, nh_in = coords_lon.shape
    nv, nc = x.shape[-2], x.shape[-1]
    n_lon, n_lat = mus_lon.shape[0], mus_lat.shape[0]
    j = seq_in * nh_in
    n_mu = n_lon * n_lat

    xt = x.reshape(n, j * nv * nc).T
    clt = coords_lon.reshape(n, j).T
    ctt = coords_lat.reshape(n, j).T
    ml16 = jnp.repeat(mus_lon, n_lat).reshape(n_mu, 1)
    mt16 = jnp.tile(mus_lat, n_lon).reshape(n_mu, 1)
    sig = jnp.asarray(sigma, jnp.float32).reshape(1, 1)

    grid = (pl.cdiv(n, _NL),)
    out = pl.pallas_call(
        _kern,
        grid=grid,
        in_specs=[
            pl.BlockSpec((j * nv * nc, _NL), lambda i: (0, i)),
            pl.BlockSpec((j, _NL), lambda i: (0, i)),
            pl.BlockSpec((j, _NL), lambda i: (0, i)),
            pl.BlockSpec((n_mu, 1), lambda i: (0, 0)),
            pl.BlockSpec((n_mu, 1), lambda i: (0, 0)),
            pl.BlockSpec((1, 1), lambda i: (0, 0)),
        ],
        out_specs=pl.BlockSpec((nv * n_mu * nc, _NL), lambda i: (0, i)),
        out_shape=jax.ShapeDtypeStruct((nv * n_mu * nc, n), jnp.float32),
        compiler_params=pltpu.CompilerParams(
            dimension_semantics=("parallel",)),
    )(xt, clt, ctt, ml16, mt16, sig)
    return out.reshape(nv, n_lon, n_lat, nc, n).transpose(4, 0, 1, 2, 3)[None]


# speedup vs baseline: 1.4832x; 1.0782x over previous
import jax
import jax.numpy as jnp
from jax.experimental import pallas as pl
from jax.experimental.pallas import tpu as pltpu

_NL = 512
_J = 12
_NV = 4
_NC = 8
_NM = 16


def _kern(x_ref, cl_ref, ct_ref, ml_ref, mt_ref, sig_ref, out_ref):
    s = jnp.maximum(sig_ref[0], 1e-10)
    h = -0.5 / (s * s)
    cl = cl_ref[...]
    ct = ct_ref[...]
    row = jax.lax.broadcasted_iota(jnp.int32, (_NM, 1), 0)
    li, ti = row // 4, row % 4
    ml = jnp.where(li == 0, ml_ref[0],
                   jnp.where(li == 1, ml_ref[1],
                             jnp.where(li == 2, ml_ref[2], ml_ref[3])))
    mt = jnp.where(ti == 0, mt_ref[0],
                   jnp.where(ti == 1, mt_ref[1],
                             jnp.where(ti == 2, mt_ref[2], mt_ref[3])))

    ws = []
    denom = None
    for j in range(_J):
        a = cl[j:j + 1, :] - ml
        b = ct[j:j + 1, :] - mt
        wj = jnp.exp(a * a * h) * jnp.exp(b * b * h)
        ws.append(wj)
        denom = wj if denom is None else denom + wj

    wn = [w / denom for w in ws]
    for v in range(_NV):
        acc = None
        for j in range(_J):
            xj = x_ref[j * 32 + v * _NC:j * 32 + (v + 1) * _NC, :]
            t = wn[j][:, None, :] * xj[None, :, :]
            acc = t if acc is None else acc + t
        out_ref[v * _NM * _NC:(v + 1) * _NM * _NC, :] = acc.reshape(
            _NM * _NC, -1)


def kernel(x, coords_lon, coords_lat, mus_lon, mus_lat, sigma):
    b, n, seq_ref, seq_in, nh_in = coords_lon.shape
    nv, nc = x.shape[-2], x.shape[-1]
    n_lon, n_lat = mus_lon.shape[0], mus_lat.shape[0]
    j = seq_in * nh_in
    n_mu = n_lon * n_lat

    xt = x.reshape(n, j * nv * nc).T
    clt = coords_lon.reshape(n, j).T
    ctt = coords_lat.reshape(n, j).T
    sig = jnp.asarray(sigma, jnp.float32).reshape(1)

    grid = (pl.cdiv(n, _NL),)
    out = pl.pallas_call(
        _kern,
        grid=grid,
        in_specs=[
            pl.BlockSpec((j * nv * nc, _NL), lambda i: (0, i)),
            pl.BlockSpec((j, _NL), lambda i: (0, i)),
            pl.BlockSpec((j, _NL), lambda i: (0, i)),
            pl.BlockSpec(memory_space=pltpu.SMEM),
            pl.BlockSpec(memory_space=pltpu.SMEM),
            pl.BlockSpec(memory_space=pltpu.SMEM),
        ],
        out_specs=pl.BlockSpec((nv * n_mu * nc, _NL), lambda i: (0, i)),
        out_shape=jax.ShapeDtypeStruct((nv * n_mu * nc, n), jnp.float32),
        compiler_params=pltpu.CompilerParams(
            dimension_semantics=("parallel",)),
    )(xt, clt, ctt, mus_lon, mus_lat, sig)
    return out.reshape(nv, n_lon, n_lat, nc, n).transpose(4, 0, 1, 2, 3)[None]


# speedup vs baseline: 1.6330x; 1.1010x over previous
import jax
import jax.numpy as jnp
from jax.experimental import pallas as pl
from jax.experimental.pallas import tpu as pltpu

_NL = 512
_J = 12
_NV = 4
_NC = 8
_NM = 16


def _kern(x_ref, cl_ref, ct_ref, ml_ref, mt_ref, sig_ref, out_ref):
    s = jnp.maximum(sig_ref[0], 1e-10)
    h = -0.5 / (s * s)
    cl3 = cl_ref[...]
    ct3 = ct_ref[...]
    row = jax.lax.broadcasted_iota(jnp.int32, (_NM, 1), 0)
    li, ti = row // 4, row % 4
    ml = jnp.where(li == 0, ml_ref[0],
                   jnp.where(li == 1, ml_ref[1],
                             jnp.where(li == 2, ml_ref[2], ml_ref[3])))
    mt = jnp.where(ti == 0, mt_ref[0],
                   jnp.where(ti == 1, mt_ref[1],
                             jnp.where(ti == 2, mt_ref[2], mt_ref[3])))

    ws = []
    denom = None
    for j in range(_J):
        sj, hj = j // 3, j % 3
        a = cl3[hj, sj][None, :] - ml
        b = ct3[hj, sj][None, :] - mt
        wj = jnp.exp(a * a * h) * jnp.exp(b * b * h)
        ws.append(wj)
        denom = wj if denom is None else denom + wj

    wn = [w / denom for w in ws]
    for v in range(_NV):
        acc = None
        for j in range(_J):
            xj = x_ref[j * 32 + v * _NC:j * 32 + (v + 1) * _NC, :]
            t = wn[j][:, None, :] * xj[None, :, :]
            acc = t if acc is None else acc + t
        out_ref[v * _NM * _NC:(v + 1) * _NM * _NC, :] = acc.reshape(
            _NM * _NC, -1)


def kernel(x, coords_lon, coords_lat, mus_lon, mus_lat, sigma):
    b, n, seq_ref, seq_in, nh_in = coords_lon.shape
    nv, nc = x.shape[-2], x.shape[-1]
    n_lon, n_lat = mus_lon.shape[0], mus_lat.shape[0]
    j = seq_in * nh_in
    n_mu = n_lon * n_lat

    xt = x.reshape(n, j * nv * nc).T
    clt = jnp.transpose(coords_lon[0, :, 0], (2, 1, 0))
    ctt = jnp.transpose(coords_lat[0, :, 0], (2, 1, 0))
    sig = jnp.asarray(sigma, jnp.float32).reshape(1)

    grid = (pl.cdiv(n, _NL),)
    out = pl.pallas_call(
        _kern,
        grid=grid,
        in_specs=[
            pl.BlockSpec((j * nv * nc, _NL), lambda i: (0, i)),
            pl.BlockSpec((nh_in, seq_in, _NL), lambda i: (0, 0, i)),
            pl.BlockSpec((nh_in, seq_in, _NL), lambda i: (0, 0, i)),
            pl.BlockSpec(memory_space=pltpu.SMEM),
            pl.BlockSpec(memory_space=pltpu.SMEM),
            pl.BlockSpec(memory_space=pltpu.SMEM),
        ],
        out_specs=pl.BlockSpec((nv * n_mu * nc, _NL), lambda i: (0, i)),
        out_shape=jax.ShapeDtypeStruct((nv * n_mu * nc, n), jnp.float32),
        compiler_params=pltpu.CompilerParams(
            dimension_semantics=("parallel",)),
    )(xt, clt, ctt, mus_lon, mus_lat, sig)
    return out.reshape(nv, n_lon, n_lat, nc, n).transpose(4, 0, 1, 2, 3)[None]


# speedup vs baseline: 1.6747x; 1.0256x over previous
import jax
import jax.numpy as jnp
from jax.experimental import pallas as pl
from jax.experimental.pallas import tpu as pltpu

_NL = 256
_J = 12
_NV = 4
_NC = 8
_NM = 16


def _kern(x_ref, cl_ref, ct_ref, ml_ref, mt_ref, sig_ref, out_ref):
    s = jnp.maximum(sig_ref[0], 1e-10)
    h = -0.5 / (s * s)
    cl3 = cl_ref[...]
    ct3 = ct_ref[...]
    row = jax.lax.broadcasted_iota(jnp.int32, (_NM, 1), 0)
    li, ti = row // 4, row % 4
    ml = jnp.where(li == 0, ml_ref[0],
                   jnp.where(li == 1, ml_ref[1],
                             jnp.where(li == 2, ml_ref[2], ml_ref[3])))
    mt = jnp.where(ti == 0, mt_ref[0],
                   jnp.where(ti == 1, mt_ref[1],
                             jnp.where(ti == 2, mt_ref[2], mt_ref[3])))

    ws = []
    denom = None
    for j in range(_J):
        sj, hj = j // 3, j % 3
        a = cl3[hj, sj][None, :] - ml
        b = ct3[hj, sj][None, :] - mt
        wj = jnp.exp(a * a * h) * jnp.exp(b * b * h)
        ws.append(wj)
        denom = wj if denom is None else denom + wj

    wn = [w / denom for w in ws]
    for v in range(_NV):
        acc = None
        for j in range(_J):
            xj = x_ref[j * 32 + v * _NC:j * 32 + (v + 1) * _NC, :]
            t = wn[j][:, None, :] * xj[None, :, :]
            acc = t if acc is None else acc + t
        out_ref[v * _NM * _NC:(v + 1) * _NM * _NC, :] = acc.reshape(
            _NM * _NC, -1)


def kernel(x, coords_lon, coords_lat, mus_lon, mus_lat, sigma):
    b, n, seq_ref, seq_in, nh_in = coords_lon.shape
    nv, nc = x.shape[-2], x.shape[-1]
    n_lon, n_lat = mus_lon.shape[0], mus_lat.shape[0]
    j = seq_in * nh_in
    n_mu = n_lon * n_lat

    xt = x.reshape(n, j * nv * nc).T
    clt = jnp.transpose(coords_lon[0, :, 0], (2, 1, 0))
    ctt = jnp.transpose(coords_lat[0, :, 0], (2, 1, 0))
    sig = jnp.asarray(sigma, jnp.float32).reshape(1)

    grid = (pl.cdiv(n, _NL),)
    out = pl.pallas_call(
        _kern,
        grid=grid,
        in_specs=[
            pl.BlockSpec((j * nv * nc, _NL), lambda i: (0, i)),
            pl.BlockSpec((nh_in, seq_in, _NL), lambda i: (0, 0, i)),
            pl.BlockSpec((nh_in, seq_in, _NL), lambda i: (0, 0, i)),
            pl.BlockSpec(memory_space=pltpu.SMEM),
            pl.BlockSpec(memory_space=pltpu.SMEM),
            pl.BlockSpec(memory_space=pltpu.SMEM),
        ],
        out_specs=pl.BlockSpec((nv * n_mu * nc, _NL), lambda i: (0, i)),
        out_shape=jax.ShapeDtypeStruct((nv * n_mu * nc, n), jnp.float32),
        compiler_params=pltpu.CompilerParams(
            dimension_semantics=("parallel",)),
    )(xt, clt, ctt, mus_lon, mus_lat, sig)
    return out.reshape(nv, n_lon, n_lat, nc, n).transpose(4, 0, 1, 2, 3)[None]


# speedup vs baseline: 2.1058x; 1.2574x over previous
import jax
import jax.numpy as jnp
from jax.experimental import pallas as pl
from jax.experimental.pallas import tpu as pltpu

_NL = 512
_J = 12
_NV = 4
_NC = 8
_NM = 16


def _kern(x_ref, cl_ref, ct_ref, ml_ref, mt_ref, sig_ref, out_ref):
    s = jnp.maximum(sig_ref[0], 1e-10)
    h = -0.5 / (s * s)
    cl3 = cl_ref[...]
    ct3 = ct_ref[...]
    row = jax.lax.broadcasted_iota(jnp.int32, (_NM, 1), 0)
    li, ti = row // 4, row % 4
    ml = jnp.where(li == 0, ml_ref[0],
                   jnp.where(li == 1, ml_ref[1],
                             jnp.where(li == 2, ml_ref[2], ml_ref[3])))
    mt = jnp.where(ti == 0, mt_ref[0],
                   jnp.where(ti == 1, mt_ref[1],
                             jnp.where(ti == 2, mt_ref[2], mt_ref[3])))

    ws = []
    denom = None
    for j in range(_J):
        sj, hj = j // 3, j % 3
        a = cl3[hj, sj][None, :] - ml
        b = ct3[hj, sj][None, :] - mt
        wj = jnp.exp(a * a * h) * jnp.exp(b * b * h)
        ws.append(wj)
        denom = wj if denom is None else denom + wj

    wn = [w / denom for w in ws]
    for v in range(_NV):
        for g in range(2):
            acc = None
            for j in range(_J):
                xj = x_ref[j * 32 + v * _NC:
                           j * 32 + (v + 1) * _NC, :]
                wh = wn[j][g * 8:(g + 1) * 8, :]
                t = wh[:, None, :] * xj[None, :, :]
                acc = t if acc is None else acc + t
            base = v * _NM * _NC + g * 8 * _NC
            out_ref[base:base + 8 * _NC, :] = acc.reshape(8 * _NC, -1)


def kernel(x, coords_lon, coords_lat, mus_lon, mus_lat, sigma):
    b, n, seq_ref, seq_in, nh_in = coords_lon.shape
    nv, nc = x.shape[-2], x.shape[-1]
    n_lon, n_lat = mus_lon.shape[0], mus_lat.shape[0]
    j = seq_in * nh_in
    n_mu = n_lon * n_lat

    xt = x.reshape(n, j * nv * nc).T
    clt = jnp.transpose(coords_lon[0, :, 0], (2, 1, 0))
    ctt = jnp.transpose(coords_lat[0, :, 0], (2, 1, 0))
    sig = jnp.asarray(sigma, jnp.float32).reshape(1)

    grid = (pl.cdiv(n, _NL),)
    out = pl.pallas_call(
        _kern,
        grid=grid,
        in_specs=[
            pl.BlockSpec((j * nv * nc, _NL), lambda i: (0, i)),
            pl.BlockSpec((nh_in, seq_in, _NL), lambda i: (0, 0, i)),
            pl.BlockSpec((nh_in, seq_in, _NL), lambda i: (0, 0, i)),
            pl.BlockSpec(memory_space=pltpu.SMEM),
            pl.BlockSpec(memory_space=pltpu.SMEM),
            pl.BlockSpec(memory_space=pltpu.SMEM),
        ],
        out_specs=pl.BlockSpec((nv * n_mu * nc, _NL), lambda i: (0, i)),
        out_shape=jax.ShapeDtypeStruct((nv * n_mu * nc, n), jnp.float32),
        compiler_params=pltpu.CompilerParams(
            dimension_semantics=("parallel",)),
    )(xt, clt, ctt, mus_lon, mus_lat, sig)
    return out.reshape(nv, n_lon, n_lat, nc, n).transpose(4, 0, 1, 2, 3)[None]


# speedup vs baseline: 2.2464x; 1.0668x over previous
import jax
import jax.numpy as jnp
from jax.experimental import pallas as pl
from jax.experimental.pallas import tpu as pltpu

_NL = 512
_J = 12
_NV = 4
_NC = 8
_NM = 16


def _kern(x_ref, cl_ref, ct_ref, ml_ref, mt_ref, sig_ref, out_ref):
    s = jnp.maximum(sig_ref[0], 1e-10)
    h = -0.5 / (s * s)
    cl3 = cl_ref[...]
    ct3 = ct_ref[...]
    row = jax.lax.broadcasted_iota(jnp.int32, (_NM, 1), 0)
    li, ti = row // 4, row % 4
    ml = jnp.where(li == 0, ml_ref[0],
                   jnp.where(li == 1, ml_ref[1],
                             jnp.where(li == 2, ml_ref[2], ml_ref[3])))
    mt = jnp.where(ti == 0, mt_ref[0],
                   jnp.where(ti == 1, mt_ref[1],
                             jnp.where(ti == 2, mt_ref[2], mt_ref[3])))

    ws = []
    denom = None
    for j in range(_J):
        sj, hj = j // 3, j % 3
        a = cl3[hj, sj][None, :] - ml
        b = ct3[hj, sj][None, :] - mt
        wj = jnp.exp(a * a * h) * jnp.exp(b * b * h)
        ws.append(wj)
        denom = wj if denom is None else denom + wj

    wn = [w / denom for w in ws]
    for v in range(_NV):
        for g in range(4):
            acc = None
            for j in range(_J):
                xj = x_ref[j * 32 + v * _NC:
                           j * 32 + (v + 1) * _NC, :]
                wh = wn[j][g * 4:(g + 1) * 4, :]
                t = wh[:, None, :] * xj[None, :, :]
                acc = t if acc is None else acc + t
            base = v * _NM * _NC + g * 4 * _NC
            out_ref[base:base + 4 * _NC, :] = acc.reshape(4 * _NC, -1)


def kernel(x, coords_lon, coords_lat, mus_lon, mus_lat, sigma):
    b, n, seq_ref, seq_in, nh_in = coords_lon.shape
    nv, nc = x.shape[-2], x.shape[-1]
    n_lon, n_lat = mus_lon.shape[0], mus_lat.shape[0]
    j = seq_in * nh_in
    n_mu = n_lon * n_lat

    xt = x.reshape(n, j * nv * nc).T
    clt = jnp.transpose(coords_lon[0, :, 0], (2, 1, 0))
    ctt = jnp.transpose(coords_lat[0, :, 0], (2, 1, 0))
    sig = jnp.asarray(sigma, jnp.float32).reshape(1)

    grid = (pl.cdiv(n, _NL),)
    out = pl.pallas_call(
        _kern,
        grid=grid,
        in_specs=[
            pl.BlockSpec((j * nv * nc, _NL), lambda i: (0, i)),
            pl.BlockSpec((nh_in, seq_in, _NL), lambda i: (0, 0, i)),
            pl.BlockSpec((nh_in, seq_in, _NL), lambda i: (0, 0, i)),
            pl.BlockSpec(memory_space=pltpu.SMEM),
            pl.BlockSpec(memory_space=pltpu.SMEM),
            pl.BlockSpec(memory_space=pltpu.SMEM),
        ],
        out_specs=pl.BlockSpec((nv * n_mu * nc, _NL), lambda i: (0, i)),
        out_shape=jax.ShapeDtypeStruct((nv * n_mu * nc, n), jnp.float32),
        compiler_params=pltpu.CompilerParams(
            dimension_semantics=("parallel",)),
    )(xt, clt, ctt, mus_lon, mus_lat, sig)
    return out.reshape(nv, n_lon, n_lat, nc, n).transpose(4, 0, 1, 2, 3)[None]


# speedup vs baseline: 2.3246x; 1.0348x over previous
import jax
import jax.numpy as jnp
from jax.experimental import pallas as pl
from jax.experimental.pallas import tpu as pltpu

_NL = 768
_J = 12
_NV = 4
_NC = 8
_NM = 16


def _kern(x_ref, cl_ref, ct_ref, ml_ref, mt_ref, sig_ref, out_ref):
    s = jnp.maximum(sig_ref[0], 1e-10)
    h = -0.5 / (s * s)
    cl3 = cl_ref[...]
    ct3 = ct_ref[...]
    row = jax.lax.broadcasted_iota(jnp.int32, (_NM, 1), 0)
    li, ti = row // 4, row % 4
    ml = jnp.where(li == 0, ml_ref[0],
                   jnp.where(li == 1, ml_ref[1],
                             jnp.where(li == 2, ml_ref[2], ml_ref[3])))
    mt = jnp.where(ti == 0, mt_ref[0],
                   jnp.where(ti == 1, mt_ref[1],
                             jnp.where(ti == 2, mt_ref[2], mt_ref[3])))

    ws = []
    denom = None
    for j in range(_J):
        sj, hj = j // 3, j % 3
        a = cl3[hj, sj][None, :] - ml
        b = ct3[hj, sj][None, :] - mt
        wj = jnp.exp(a * a * h) * jnp.exp(b * b * h)
        ws.append(wj)
        denom = wj if denom is None else denom + wj

    wn = [w / denom for w in ws]
    for v in range(_NV):
        for g in range(4):
            acc = None
            for j in range(_J):
                xj = x_ref[j * 32 + v * _NC:
                           j * 32 + (v + 1) * _NC, :]
                wh = wn[j][g * 4:(g + 1) * 4, :]
                t = wh[:, None, :] * xj[None, :, :]
                acc = t if acc is None else acc + t
            base = v * _NM * _NC + g * 4 * _NC
            out_ref[base:base + 4 * _NC, :] = acc.reshape(4 * _NC, -1)


def kernel(x, coords_lon, coords_lat, mus_lon, mus_lat, sigma):
    b, n, seq_ref, seq_in, nh_in = coords_lon.shape
    nv, nc = x.shape[-2], x.shape[-1]
    n_lon, n_lat = mus_lon.shape[0], mus_lat.shape[0]
    j = seq_in * nh_in
    n_mu = n_lon * n_lat

    xt = x.reshape(n, j * nv * nc).T
    clt = jnp.transpose(coords_lon[0, :, 0], (2, 1, 0))
    ctt = jnp.transpose(coords_lat[0, :, 0], (2, 1, 0))
    sig = jnp.asarray(sigma, jnp.float32).reshape(1)

    grid = (pl.cdiv(n, _NL),)
    out = pl.pallas_call(
        _kern,
        grid=grid,
        in_specs=[
            pl.BlockSpec((j * nv * nc, _NL), lambda i: (0, i)),
            pl.BlockSpec((nh_in, seq_in, _NL), lambda i: (0, 0, i)),
            pl.BlockSpec((nh_in, seq_in, _NL), lambda i: (0, 0, i)),
            pl.BlockSpec(memory_space=pltpu.SMEM),
            pl.BlockSpec(memory_space=pltpu.SMEM),
            pl.BlockSpec(memory_space=pltpu.SMEM),
        ],
        out_specs=pl.BlockSpec((nv * n_mu * nc, _NL), lambda i: (0, i)),
        out_shape=jax.ShapeDtypeStruct((nv * n_mu * nc, n), jnp.float32),
        compiler_params=pltpu.CompilerParams(
            dimension_semantics=("parallel",)),
    )(xt, clt, ctt, mus_lon, mus_lat, sig)
    return out.reshape(nv, n_lon, n_lat, nc, n).transpose(4, 0, 1, 2, 3)[None]
